# Initial kernel scaffold; baseline (speedup 1.0000x reference)
import jax
import jax.numpy as jnp
from jax.experimental import pallas as pl


def kernel(points, q_coords, support_idxs, cy_params, disk_params, cone_params, lambdas):
    raise NotImplementedError("write your pallas kernel here")


# speedup vs baseline: 3.9069x; 3.9069x over previous
import functools

import jax
import jax.numpy as jnp
from jax import lax
from jax.experimental import pallas as pl
from jax.experimental.pallas import tpu as pltpu
from jax.experimental.pallas import tpu_sc as plsc

N_POINTS = 100000
M_QUERIES = 50000
K_SUPPORT = 16
G_CY = 8
G_DISK = 8
G_CONE = 8
NUM_GIBS = G_CY + G_DISK + G_CONE
NUM_OBS = 16
KERNEL_REACH = 0.3
REACH2 = KERNEL_REACH * KERNEL_REACH

_SC_CORES = 2
_SC_SUBCORES = 16
_NW = _SC_CORES * _SC_SUBCORES

_BM = 512
_MP = 50176
_BP = _MP * K_SUPPORT
_B_PER_W = _BP // _NW
_CHUNK = _B_PER_W


def _sc_gather_body(px, py, pz, idx_hbm, sx_hbm, sy_hbm, sz_hbm,
                    idx_v, vx, vy, vz, sem):
  wid = lax.axis_index("s") * _SC_CORES + lax.axis_index("c")
  base = wid * _B_PER_W
  pltpu.sync_copy(idx_hbm.at[pl.ds(base, _CHUNK)], idx_v)
  for tab, val, out in ((px, vx, sx_hbm), (py, vy, sy_hbm), (pz, vz, sz_hbm)):
    pltpu.async_copy(tab.at[idx_v], val, sem).wait()
    pltpu.sync_copy(val, out.at[pl.ds(base, _CHUNK)])


def _sc_gather(px, py, pz, idx_flat):
  mesh = plsc.VectorSubcoreMesh(core_axis_name="c", subcore_axis_name="s")
  flat = jax.ShapeDtypeStruct((_BP,), jnp.float32)
  return pl.kernel(
      _sc_gather_body,
      out_type=(flat, flat, flat),
      mesh=mesh,
      scratch_types=[
          pltpu.VMEM((_CHUNK,), jnp.int32),
          pltpu.VMEM((_CHUNK,), jnp.float32),
          pltpu.VMEM((_CHUNK,), jnp.float32),
          pltpu.VMEM((_CHUNK,), jnp.float32),
          pltpu.SemaphoreType.DMA,
      ],
      compiler_params=pltpu.CompilerParams(use_tc_tiling_on_sc=False),
  )(px, py, pz, idx_flat)


def _tc_body(s_ref, q_ref, lam_ref, par_ref, out_ref, gib_ref):
  K = K_SUPPORT
  sx = s_ref[0:K, :]
  sy = s_ref[K:2 * K, :]
  sz = s_ref[2 * K:3 * K, :]
  rx = sx - q_ref[0:1, :]
  ry = sy - q_ref[1:2, :]
  rz = sz - q_ref[2:3, :]
  x2y2 = rx * rx + ry * ry
  z2 = rz * rz
  d2 = x2y2 + z2
  mask = d2 <= REACH2
  zero = jnp.zeros_like(x2y2)

  for g in range(G_CY):
    a = par_ref[g]
    v = jnp.where(mask, jnp.exp(x2y2 * (-a)), zero)
    gib_ref[g:g + 1, :] = jnp.sum(v, axis=0, keepdims=True)
  for g in range(G_DISK):
    a = par_ref[G_CY + g]
    b = par_ref[G_CY + G_DISK + g]
    v = jnp.where(mask, jnp.exp(-(x2y2 * a + z2 * b)), zero)
    gib_ref[G_CY + g:G_CY + g + 1, :] = jnp.sum(v, axis=0, keepdims=True)
  for g in range(G_CONE):
    cr = par_ref[G_CY + 2 * G_DISK + g]
    ci = par_ref[G_CY + 2 * G_DISK + G_CONE + g]
    r = jnp.maximum(cr * (1.0 + ci * rz), 1e-3)
    v = jnp.where(mask, jnp.exp(x2y2 * (-0.5) / (r * r)), zero)
    gib_ref[G_CY + G_DISK + g:G_CY + G_DISK + g + 1, :] = jnp.sum(
        v, axis=0, keepdims=True)

  out_ref[...] = lax.dot_general(
      gib_ref[...], lam_ref[...],
      dimension_numbers=(((0,), (0,)), ((), ())),
      preferred_element_type=jnp.float32)


def _tc_compute(sT, q8, lambdas, params):
  return pl.pallas_call(
      _tc_body,
      grid=(_MP // _BM,),
      in_specs=[
          pl.BlockSpec((3 * K_SUPPORT, _BM), lambda i: (0, i)),
          pl.BlockSpec((8, _BM), lambda i: (0, i)),
          pl.BlockSpec((NUM_GIBS, NUM_OBS), lambda i: (0, 0)),
          pl.BlockSpec(memory_space=pltpu.SMEM),
      ],
      out_specs=pl.BlockSpec((_BM, NUM_OBS), lambda i: (i, 0)),
      out_shape=jax.ShapeDtypeStruct((_MP, NUM_OBS), jnp.float32),
      scratch_shapes=[pltpu.VMEM((NUM_GIBS, _BM), jnp.float32)],
  )(sT, q8, lambdas, params)


def kernel(points, q_coords, support_idxs, cy_params, disk_params,
           cone_params, lambdas):
  idx = support_idxs.astype(jnp.int32)
  pts = points.astype(jnp.float32)
  idxp = jnp.pad(idx, ((0, _MP - M_QUERIES), (0, 0))).reshape(-1)
  sx, sy, sz = _sc_gather(pts[:, 0], pts[:, 1], pts[:, 2], idxp)

  sT = jnp.concatenate(
      [a.reshape(_MP, K_SUPPORT).T for a in (sx, sy, sz)], axis=0)
  qT = jnp.pad(q_coords.astype(jnp.float32),
               ((0, _MP - M_QUERIES), (0, 0))).T
  q8 = jnp.pad(qT, ((0, 5), (0, 0)))

  cy_a = 0.5 / (cy_params * cy_params)
  disk_a = 0.5 / (disk_params[:, 0] * disk_params[:, 0])
  disk_b = 0.5 / (disk_params[:, 1] * disk_params[:, 1])
  params = jnp.concatenate(
      [cy_a, disk_a, disk_b, cone_params[:, 0], cone_params[:, 1]]
  ).astype(jnp.float32)

  out = _tc_compute(sT, q8, lambdas.astype(jnp.float32), params)
  return out[:M_QUERIES]


# speedup vs baseline: 4.5512x; 1.1649x over previous
import functools

import jax
import jax.numpy as jnp
from jax import lax
from jax.experimental import pallas as pl
from jax.experimental.pallas import tpu as pltpu
from jax.experimental.pallas import tpu_sc as plsc

N_POINTS = 100000
M_QUERIES = 50000
K_SUPPORT = 16
G_CY = 8
G_DISK = 8
G_CONE = 8
NUM_GIBS = G_CY + G_DISK + G_CONE
NUM_OBS = 16
KERNEL_REACH = 0.3
REACH2 = KERNEL_REACH * KERNEL_REACH

_SC_CORES = 2
_SC_SUBCORES = 16
_NW = _SC_CORES * _SC_SUBCORES

_BM = 512
_MP = 50176
_BP = _MP * K_SUPPORT
_B_PER_W = _BP // _NW
_Q_PER_W = _MP // _NW
_CHQ = 784
_CHUNK = _CHQ * K_SUPPORT
_N_CHUNKS = _Q_PER_W // _CHQ


def _sc_gather_body(px, py, pz, qx, qy, qz, idx_hbm,
                    rx_hbm, ry_hbm, rz_hbm,
                    tab_v, idx_v, val_v, q_v, sem):
  wid = lax.axis_index("s") * _SC_CORES + lax.axis_index("c")
  qbase = wid * _Q_PER_W
  ebase = wid * _B_PER_W
  for tab, q, out in ((px, qx, rx_hbm), (py, qy, ry_hbm), (pz, qz, rz_hbm)):
    pltpu.sync_copy(tab, tab_v)
    pltpu.sync_copy(q.at[pl.ds(qbase, _Q_PER_W)], q_v)
    for c in range(_N_CHUNKS):
      pltpu.sync_copy(idx_hbm.at[pl.ds(ebase + c * _CHUNK, _CHUNK)], idx_v)

      def body(jq, _, c=c):
        qv = q_v[pl.ds((c * _CHQ // 16 + jq) * 16, 16)]
        for jj in range(16):
          e = (jq * 16 + jj) * 16
          iv = idx_v[pl.ds(e, 16)]
          s = plsc.load_gather(tab_v, [iv])
          val_v[pl.ds(e, 16)] = s - qv[jj]
        return ()

      lax.fori_loop(0, _CHQ // 16, body, (), unroll=2)
      pltpu.sync_copy(val_v, out.at[pl.ds(ebase + c * _CHUNK, _CHUNK)])


def _sc_gather(px, py, pz, qx, qy, qz, idx_flat):
  mesh = plsc.VectorSubcoreMesh(core_axis_name="c", subcore_axis_name="s")
  flat = jax.ShapeDtypeStruct((_BP,), jnp.float32)
  return pl.kernel(
      _sc_gather_body,
      out_type=(flat, flat, flat),
      mesh=mesh,
      scratch_types=[
          pltpu.VMEM((N_POINTS,), jnp.float32),
          pltpu.VMEM((_CHUNK,), jnp.int32),
          pltpu.VMEM((_CHUNK,), jnp.float32),
          pltpu.VMEM((_Q_PER_W,), jnp.float32),
          pltpu.SemaphoreType.DMA,
      ],
      compiler_params=pltpu.CompilerParams(
          use_tc_tiling_on_sc=False, needs_layout_passes=False),
  )(px, py, pz, qx, qy, qz, idx_flat)


def _tc_body(s_ref, lam_ref, par_ref, out_ref, gib_ref):
  K = K_SUPPORT
  rx = s_ref[0:K, :]
  ry = s_ref[K:2 * K, :]
  rz = s_ref[2 * K:3 * K, :]
  x2y2 = rx * rx + ry * ry
  z2 = rz * rz
  d2 = x2y2 + z2
  mask = d2 <= REACH2
  zero = jnp.zeros_like(x2y2)

  for g in range(G_CY):
    a = par_ref[g]
    v = jnp.where(mask, jnp.exp(x2y2 * (-a)), zero)
    gib_ref[g:g + 1, :] = jnp.sum(v, axis=0, keepdims=True)
  for g in range(G_DISK):
    a = par_ref[G_CY + g]
    b = par_ref[G_CY + G_DISK + g]
    v = jnp.where(mask, jnp.exp(-(x2y2 * a + z2 * b)), zero)
    gib_ref[G_CY + g:G_CY + g + 1, :] = jnp.sum(v, axis=0, keepdims=True)
  for g in range(G_CONE):
    cr = par_ref[G_CY + 2 * G_DISK + g]
    ci = par_ref[G_CY + 2 * G_DISK + G_CONE + g]
    r = jnp.maximum(cr * (1.0 + ci * rz), 1e-3)
    v = jnp.where(mask, jnp.exp(x2y2 * (-0.5) / (r * r)), zero)
    gib_ref[G_CY + G_DISK + g:G_CY + G_DISK + g + 1, :] = jnp.sum(
        v, axis=0, keepdims=True)

  out_ref[...] = lax.dot_general(
      gib_ref[...], lam_ref[...],
      dimension_numbers=(((0,), (0,)), ((), ())),
      preferred_element_type=jnp.float32)


def _tc_compute(sT, lambdas, params):
  return pl.pallas_call(
      _tc_body,
      grid=(_MP // _BM,),
      in_specs=[
          pl.BlockSpec((3 * K_SUPPORT, _BM), lambda i: (0, i)),
          pl.BlockSpec((NUM_GIBS, NUM_OBS), lambda i: (0, 0)),
          pl.BlockSpec(memory_space=pltpu.SMEM),
      ],
      out_specs=pl.BlockSpec((_BM, NUM_OBS), lambda i: (i, 0)),
      out_shape=jax.ShapeDtypeStruct((_MP, NUM_OBS), jnp.float32),
      scratch_shapes=[pltpu.VMEM((NUM_GIBS, _BM), jnp.float32)],
  )(sT, lambdas, params)


def kernel(points, q_coords, support_idxs, cy_params, disk_params,
           cone_params, lambdas):
  idx = support_idxs.astype(jnp.int32)
  pts = points.astype(jnp.float32)
  qp = jnp.pad(q_coords.astype(jnp.float32), ((0, _MP - M_QUERIES), (0, 0)))
  idxp = jnp.pad(idx, ((0, _MP - M_QUERIES), (0, 0))).reshape(-1)
  rx, ry, rz = _sc_gather(pts[:, 0], pts[:, 1], pts[:, 2],
                          qp[:, 0], qp[:, 1], qp[:, 2], idxp)

  sT = jnp.concatenate(
      [a.reshape(_MP, K_SUPPORT).T for a in (rx, ry, rz)], axis=0)

  cy_a = 0.5 / (cy_params * cy_params)
  disk_a = 0.5 / (disk_params[:, 0] * disk_params[:, 0])
  disk_b = 0.5 / (disk_params[:, 1] * disk_params[:, 1])
  params = jnp.concatenate(
      [cy_a, disk_a, disk_b, cone_params[:, 0], cone_params[:, 1]]
  ).astype(jnp.float32)

  out = _tc_compute(sT, lambdas.astype(jnp.float32), params)
  return out[:M_QUERIES]


# speedup vs baseline: 6.1948x; 1.3611x over previous
import functools

import jax
import jax.numpy as jnp
from jax import lax
from jax.experimental import pallas as pl
from jax.experimental.pallas import tpu as pltpu
from jax.experimental.pallas import tpu_sc as plsc

N_POINTS = 100000
M_QUERIES = 50000
K_SUPPORT = 16
G_CY = 8
G_DISK = 8
G_CONE = 8
NUM_GIBS = G_CY + G_DISK + G_CONE
NUM_OBS = 16
KERNEL_REACH = 0.3
REACH2 = KERNEL_REACH * KERNEL_REACH

_SC_CORES = 2
_SC_SUBCORES = 16
_NW = _SC_CORES * _SC_SUBCORES

_BM = 512
_MP = 50176
_BP = _MP * K_SUPPORT
_B_PER_W = _BP // _NW
_Q_PER_W = _MP // _NW
_CHQ = 784
_CHUNK = _CHQ * K_SUPPORT
_N_CHUNKS = _Q_PER_W // _CHQ


def _sc_gather_body(px, py, pz, qx, qy, qz, idx_hbm, out_hbm,
                    tab_v, idx_v, kt_v, q_v, sem):
  wid = lax.axis_index("s") * _SC_CORES + lax.axis_index("c")
  qbase = wid * _Q_PER_W
  ebase = wid * _B_PER_W
  iota = lax.iota(jnp.int32, 16)
  for ci, (tab, q) in enumerate(((px, qx), (py, qy), (pz, qz))):
    pltpu.sync_copy(tab, tab_v)
    pltpu.sync_copy(q.at[pl.ds(qbase, _Q_PER_W)], q_v)
    for c in range(_N_CHUNKS):
      pltpu.sync_copy(idx_hbm.at[pl.ds(ebase + c * _CHUNK, _CHUNK)], idx_v)

      def body(jq, _, c=c):
        qv = q_v[pl.ds((c * _CHQ // 16 + jq) * 16, 16)]
        for jj in range(16):
          e = (jq * 16 + jj) * 16
          iv = idx_v[pl.ds(e, 16)]
          s = plsc.load_gather(tab_v, [iv])
          col = jnp.full((16,), jq * 16 + jj, jnp.int32)
          plsc.store_scatter(kt_v, [iota, col], s - qv[jj])
        return ()

      lax.fori_loop(0, _CHQ // 16, body, (), unroll=2)
      cps = []
      for k in range(K_SUPPORT):
        off = (ci * K_SUPPORT + k) * _MP + qbase + c * _CHQ
        cps.append(pltpu.async_copy(kt_v.at[k], out_hbm.at[pl.ds(off, _CHQ)],
                                    sem))
      for cp in cps:
        cp.wait()


def _sc_gather(px, py, pz, qx, qy, qz, idx_flat):
  mesh = plsc.VectorSubcoreMesh(core_axis_name="c", subcore_axis_name="s")
  return pl.kernel(
      _sc_gather_body,
      out_type=jax.ShapeDtypeStruct((3 * K_SUPPORT * _MP,), jnp.float32),
      mesh=mesh,
      scratch_types=[
          pltpu.VMEM((N_POINTS,), jnp.float32),
          pltpu.VMEM((_CHUNK,), jnp.int32),
          pltpu.VMEM((K_SUPPORT, _CHQ), jnp.float32),
          pltpu.VMEM((_Q_PER_W,), jnp.float32),
          pltpu.SemaphoreType.DMA,
      ],
      compiler_params=pltpu.CompilerParams(
          use_tc_tiling_on_sc=False, needs_layout_passes=False),
  )(px, py, pz, qx, qy, qz, idx_flat)


def _tc_body(s_ref, lam_ref, par_ref, out_ref, gib_ref):
  K = K_SUPPORT
  rx = s_ref[0:K, :]
  ry = s_ref[K:2 * K, :]
  rz = s_ref[2 * K:3 * K, :]
  x2y2 = rx * rx + ry * ry
  z2 = rz * rz
  d2 = x2y2 + z2
  mask = d2 <= REACH2
  zero = jnp.zeros_like(x2y2)

  for g in range(G_CY):
    a = par_ref[g]
    v = jnp.where(mask, jnp.exp(x2y2 * (-a)), zero)
    gib_ref[g:g + 1, :] = jnp.sum(v, axis=0, keepdims=True)
  for g in range(G_DISK):
    a = par_ref[G_CY + g]
    b = par_ref[G_CY + G_DISK + g]
    v = jnp.where(mask, jnp.exp(-(x2y2 * a + z2 * b)), zero)
    gib_ref[G_CY + g:G_CY + g + 1, :] = jnp.sum(v, axis=0, keepdims=True)
  for g in range(G_CONE):
    cr = par_ref[G_CY + 2 * G_DISK + g]
    ci = par_ref[G_CY + 2 * G_DISK + G_CONE + g]
    r = jnp.maximum(cr * (1.0 + ci * rz), 1e-3)
    v = jnp.where(mask, jnp.exp(x2y2 * (-0.5) / (r * r)), zero)
    gib_ref[G_CY + G_DISK + g:G_CY + G_DISK + g + 1, :] = jnp.sum(
        v, axis=0, keepdims=True)

  out_ref[...] = lax.dot_general(
      gib_ref[...], lam_ref[...],
      dimension_numbers=(((0,), (0,)), ((), ())),
      preferred_element_type=jnp.float32)


def _tc_compute(sT, lambdas, params):
  return pl.pallas_call(
      _tc_body,
      grid=(_MP // _BM,),
      in_specs=[
          pl.BlockSpec((3 * K_SUPPORT, _BM), lambda i: (0, i)),
          pl.BlockSpec((NUM_GIBS, NUM_OBS), lambda i: (0, 0)),
          pl.BlockSpec(memory_space=pltpu.SMEM),
      ],
      out_specs=pl.BlockSpec((_BM, NUM_OBS), lambda i: (i, 0)),
      out_shape=jax.ShapeDtypeStruct((M_QUERIES, NUM_OBS), jnp.float32),
      scratch_shapes=[pltpu.VMEM((NUM_GIBS, _BM), jnp.float32)],
  )(sT, lambdas, params)


def kernel(points, q_coords, support_idxs, cy_params, disk_params,
           cone_params, lambdas):
  idx = support_idxs.astype(jnp.int32)
  pts = points.astype(jnp.float32)
  qp = jnp.pad(q_coords.astype(jnp.float32), ((0, _MP - M_QUERIES), (0, 0)))
  idxp = jnp.pad(idx, ((0, _MP - M_QUERIES), (0, 0))).reshape(-1)
  flat = _sc_gather(pts[:, 0], pts[:, 1], pts[:, 2],
                    qp[:, 0], qp[:, 1], qp[:, 2], idxp)
  sT = flat.reshape(3 * K_SUPPORT, _MP)

  cy_a = 0.5 / (cy_params * cy_params)
  disk_a = 0.5 / (disk_params[:, 0] * disk_params[:, 0])
  disk_b = 0.5 / (disk_params[:, 1] * disk_params[:, 1])
  params = jnp.concatenate(
      [cy_a, disk_a, disk_b, cone_params[:, 0], cone_params[:, 1]]
  ).astype(jnp.float32)

  return _tc_compute(sT, lambdas.astype(jnp.float32), params)


# speedup vs baseline: 6.4770x; 1.0456x over previous
import functools

import jax
import jax.numpy as jnp
from jax import lax
from jax.experimental import pallas as pl
from jax.experimental.pallas import tpu as pltpu
from jax.experimental.pallas import tpu_sc as plsc

N_POINTS = 100000
M_QUERIES = 50000
K_SUPPORT = 16
G_CY = 8
G_DISK = 8
G_CONE = 8
NUM_GIBS = G_CY + G_DISK + G_CONE
NUM_OBS = 16
KERNEL_REACH = 0.3
REACH2 = KERNEL_REACH * KERNEL_REACH

_SC_CORES = 2
_SC_SUBCORES = 16
_NW = _SC_CORES * _SC_SUBCORES

_BM = 512
_MP = 50176
_BP = _MP * K_SUPPORT
_B_PER_W = _BP // _NW
_Q_PER_W = _MP // _NW
_CHQ = 784
_CHUNK = _CHQ * K_SUPPORT
_N_CHUNKS = _Q_PER_W // _CHQ


def _sc_gather_body(pts_cm, q_cm, idx_hbm, out_hbm,
                    tab_v, idx_v, kt_v, q_v, sem):
  wid = lax.axis_index("s") * _SC_CORES + lax.axis_index("c")
  qbase = wid * _Q_PER_W
  ebase = wid * _B_PER_W
  iota = lax.iota(jnp.int32, 16)
  for ci in range(3):
    pltpu.sync_copy(pts_cm.at[pl.ds(ci * N_POINTS, N_POINTS)], tab_v)
    pltpu.sync_copy(q_cm.at[pl.ds(ci * _MP + qbase, _Q_PER_W)], q_v)
    for c in range(_N_CHUNKS):
      pltpu.sync_copy(idx_hbm.at[pl.ds(ebase + c * _CHUNK, _CHUNK)], idx_v)

      def body(jq, _, c=c):
        qv = q_v[pl.ds((c * _CHQ // 16 + jq) * 16, 16)]
        for jj in range(16):
          e = (jq * 16 + jj) * 16
          iv = idx_v[pl.ds(e, 16)]
          s = plsc.load_gather(tab_v, [iv])
          col = jnp.full((16,), jq * 16 + jj, jnp.int32)
          plsc.store_scatter(kt_v, [iota, col], s - qv[jj])
        return ()

      lax.fori_loop(0, _CHQ // 16, body, (), unroll=2)
      cps = []
      for k in range(K_SUPPORT):
        off = (ci * K_SUPPORT + k) * _MP + qbase + c * _CHQ
        cps.append(pltpu.async_copy(kt_v.at[k], out_hbm.at[pl.ds(off, _CHQ)],
                                    sem))
      for cp in cps:
        cp.wait()


def _sc_gather(pts_cm, q_cm, idx_flat):
  mesh = plsc.VectorSubcoreMesh(core_axis_name="c", subcore_axis_name="s")
  return pl.kernel(
      _sc_gather_body,
      out_type=jax.ShapeDtypeStruct((3 * K_SUPPORT * _MP,), jnp.float32),
      mesh=mesh,
      scratch_types=[
          pltpu.VMEM((N_POINTS,), jnp.float32),
          pltpu.VMEM((_CHUNK,), jnp.int32),
          pltpu.VMEM((K_SUPPORT, _CHQ), jnp.float32),
          pltpu.VMEM((_Q_PER_W,), jnp.float32),
          pltpu.SemaphoreType.DMA,
      ],
      compiler_params=pltpu.CompilerParams(
          use_tc_tiling_on_sc=False, needs_layout_passes=False),
  )(pts_cm, q_cm, idx_flat)


def _tc_body(s_ref, w_ref, par_ref, out_ref, v_ref):
  K = K_SUPPORT
  rx = s_ref[0:K, :]
  ry = s_ref[K:2 * K, :]
  rz = s_ref[2 * K:3 * K, :]
  x2y2 = rx * rx + ry * ry
  z2 = rz * rz
  mask = x2y2 + z2 <= REACH2
  zero = jnp.zeros_like(x2y2)
  nx = x2y2 * (-0.5)

  for g in range(G_CY):
    a = par_ref[g]
    v_ref[pl.ds(g * K, K), :] = jnp.where(
        mask, jnp.exp(x2y2 * (-a)), zero)
  for g in range(G_DISK):
    a = par_ref[G_CY + g]
    b = par_ref[G_CY + G_DISK + g]
    v_ref[pl.ds((G_CY + g) * K, K), :] = jnp.where(
        mask, jnp.exp(-(x2y2 * a + z2 * b)), zero)
  for g in range(G_CONE):
    cr = par_ref[G_CY + 2 * G_DISK + g]
    ci = par_ref[G_CY + 2 * G_DISK + G_CONE + g]
    r = jnp.maximum(cr * (1.0 + ci * rz), 1e-3)
    inv = pl.reciprocal(r, approx=True)
    v_ref[pl.ds((G_CY + G_DISK + g) * K, K), :] = jnp.where(
        mask, jnp.exp(nx * (inv * inv)), zero)

  out_ref[...] = lax.dot_general(
      v_ref[...], w_ref[...],
      dimension_numbers=(((0,), (1,)), ((), ())),
      preferred_element_type=jnp.float32)


def _tc_compute(sT, w, params):
  return pl.pallas_call(
      _tc_body,
      grid=(_MP // _BM,),
      in_specs=[
          pl.BlockSpec((3 * K_SUPPORT, _BM), lambda i: (0, i)),
          pl.BlockSpec((NUM_OBS, NUM_GIBS * K_SUPPORT), lambda i: (0, 0)),
          pl.BlockSpec(memory_space=pltpu.SMEM),
      ],
      out_specs=pl.BlockSpec((_BM, NUM_OBS), lambda i: (i, 0)),
      out_shape=jax.ShapeDtypeStruct((M_QUERIES, NUM_OBS), jnp.float32),
      scratch_shapes=[pltpu.VMEM((NUM_GIBS * K_SUPPORT, _BM), jnp.float32)],
  )(sT, w, params)


def kernel(points, q_coords, support_idxs, cy_params, disk_params,
           cone_params, lambdas):
  idx = support_idxs.astype(jnp.int32)
  pts_cm = points.astype(jnp.float32).T.reshape(-1)
  q_cm = jnp.pad(q_coords.astype(jnp.float32),
                 ((0, _MP - M_QUERIES), (0, 0))).T.reshape(-1)
  idxp = jnp.pad(idx, ((0, _MP - M_QUERIES), (0, 0))).reshape(-1)
  flat = _sc_gather(pts_cm, q_cm, idxp)
  sT = flat.reshape(3 * K_SUPPORT, _MP)

  cy_a = 0.5 / (cy_params * cy_params)
  disk_a = 0.5 / (disk_params[:, 0] * disk_params[:, 0])
  disk_b = 0.5 / (disk_params[:, 1] * disk_params[:, 1])
  params = jnp.concatenate(
      [cy_a, disk_a, disk_b, cone_params[:, 0], cone_params[:, 1]]
  ).astype(jnp.float32)

  w = jnp.repeat(lambdas.astype(jnp.float32).T, K_SUPPORT, axis=1)
  return _tc_compute(sT, w, params)


# speedup vs baseline: 7.8180x; 1.2070x over previous
import functools

import jax
import jax.numpy as jnp
from jax import lax
from jax.experimental import pallas as pl
from jax.experimental.pallas import tpu as pltpu
from jax.experimental.pallas import tpu_sc as plsc

N_POINTS = 100000
M_QUERIES = 50000
K_SUPPORT = 16
G_CY = 8
G_DISK = 8
G_CONE = 8
NUM_GIBS = G_CY + G_DISK + G_CONE
NUM_OBS = 16
KERNEL_REACH = 0.3
REACH2 = KERNEL_REACH * KERNEL_REACH

_SC_CORES = 2
_SC_SUBCORES = 16
_NW = _SC_CORES * _SC_SUBCORES

_BM = 1024
_MP = 50176
_BP = _MP * K_SUPPORT
_B_PER_W = _BP // _NW
_Q_PER_W = _MP // _NW
_CHQ = 784
_CHUNK = _CHQ * K_SUPPORT
_N_CHUNKS = _Q_PER_W // _CHQ
_B_REAL = M_QUERIES * K_SUPPORT
_TAIL = _B_REAL - (_NW - 1) * _B_PER_W - (_N_CHUNKS - 1) * _CHUNK


def _sc_gather_body(pts_cm, q_cm, idx_hbm, out_hbm,
                    tab_v, idx_v, kt_v, q_v, sem):
  wid = lax.axis_index("s") * _SC_CORES + lax.axis_index("c")
  qbase = wid * _Q_PER_W
  ebase = wid * _B_PER_W
  iota = lax.iota(jnp.int32, 16)
  for ci in range(3):
    pltpu.sync_copy(pts_cm.at[pl.ds(ci * N_POINTS, N_POINTS)], tab_v)
    pltpu.sync_copy(q_cm.at[pl.ds(ci * _MP + qbase, _Q_PER_W)], q_v)
    for c in range(_N_CHUNKS):
      if c == _N_CHUNKS - 1:
        @pl.when(wid == _NW - 1)
        def _():
          pltpu.sync_copy(idx_hbm.at[pl.ds(ebase + c * _CHUNK, _TAIL)],
                          idx_v.at[pl.ds(0, _TAIL)])

        @pl.when(wid < _NW - 1)
        def _():
          pltpu.sync_copy(idx_hbm.at[pl.ds(ebase + c * _CHUNK, _CHUNK)],
                          idx_v)
      else:
        pltpu.sync_copy(idx_hbm.at[pl.ds(ebase + c * _CHUNK, _CHUNK)], idx_v)

      def body(jq, _, c=c):
        qv = q_v[pl.ds((c * _CHQ // 16 + jq) * 16, 16)]
        for jj in range(16):
          e = (jq * 16 + jj) * 16
          iv = idx_v[pl.ds(e, 16)]
          s = plsc.load_gather(tab_v, [iv])
          col = jnp.full((16,), jq * 16 + jj, jnp.int32)
          plsc.store_scatter(kt_v, [iota, col], s - qv[jj])
        return ()

      lax.fori_loop(0, _CHQ // 16, body, (), unroll=2)
      cps = []
      for k in range(K_SUPPORT):
        off = (ci * K_SUPPORT + k) * _MP + qbase + c * _CHQ
        cps.append(pltpu.async_copy(kt_v.at[k], out_hbm.at[pl.ds(off, _CHQ)],
                                    sem))
      for cp in cps:
        cp.wait()


def _sc_gather(pts_cm, q_cm, idx_flat):
  mesh = plsc.VectorSubcoreMesh(core_axis_name="c", subcore_axis_name="s")
  return pl.kernel(
      _sc_gather_body,
      out_type=jax.ShapeDtypeStruct((3 * K_SUPPORT * _MP,), jnp.float32),
      mesh=mesh,
      scratch_types=[
          pltpu.VMEM((N_POINTS,), jnp.float32),
          pltpu.VMEM((_CHUNK,), jnp.int32),
          pltpu.VMEM((K_SUPPORT, _CHQ), jnp.float32),
          pltpu.VMEM((_Q_PER_W,), jnp.float32),
          pltpu.SemaphoreType.DMA,
      ],
      compiler_params=pltpu.CompilerParams(
          use_tc_tiling_on_sc=False, needs_layout_passes=False),
  )(pts_cm, q_cm, idx_flat)


def _tc_body(s_ref, w_ref, par_ref, out_ref, v_ref):
  K = K_SUPPORT
  rx = s_ref[0:K, :]
  ry = s_ref[K:2 * K, :]
  rz = s_ref[2 * K:3 * K, :]
  x2y2 = rx * rx + ry * ry
  z2 = rz * rz
  mask = x2y2 + z2 <= REACH2
  zero = jnp.zeros_like(x2y2)
  nx = x2y2 * (-0.5)

  for g in range(G_CY):
    a = par_ref[g]
    v_ref[pl.ds(g * K, K), :] = jnp.where(
        mask, jnp.exp(x2y2 * (-a)), zero)
  for g in range(G_DISK):
    a = par_ref[G_CY + g]
    b = par_ref[G_CY + G_DISK + g]
    v_ref[pl.ds((G_CY + g) * K, K), :] = jnp.where(
        mask, jnp.exp(-(x2y2 * a + z2 * b)), zero)
  for g in range(G_CONE):
    cr = par_ref[G_CY + 2 * G_DISK + g]
    ci = par_ref[G_CY + 2 * G_DISK + G_CONE + g]
    r = jnp.maximum(cr * (1.0 + ci * rz), 1e-3)
    inv = pl.reciprocal(r, approx=True)
    v_ref[pl.ds((G_CY + G_DISK + g) * K, K), :] = jnp.where(
        mask, jnp.exp(nx * (inv * inv)), zero)

  out_ref[...] = lax.dot_general(
      v_ref[...], w_ref[...],
      dimension_numbers=(((0,), (1,)), ((), ())),
      preferred_element_type=jnp.float32)


def _tc_compute(sT, w, params):
  return pl.pallas_call(
      _tc_body,
      grid=(_MP // _BM,),
      in_specs=[
          pl.BlockSpec((3 * K_SUPPORT, _BM), lambda i: (0, i)),
          pl.BlockSpec((NUM_OBS, NUM_GIBS * K_SUPPORT), lambda i: (0, 0)),
          pl.BlockSpec(memory_space=pltpu.SMEM),
      ],
      out_specs=pl.BlockSpec((_BM, NUM_OBS), lambda i: (i, 0)),
      out_shape=jax.ShapeDtypeStruct((M_QUERIES, NUM_OBS), jnp.float32),
      scratch_shapes=[pltpu.VMEM((NUM_GIBS * K_SUPPORT, _BM), jnp.float32)],
  )(sT, w, params)


def kernel(points, q_coords, support_idxs, cy_params, disk_params,
           cone_params, lambdas):
  idx = support_idxs.astype(jnp.int32)
  pts_cm = points.astype(jnp.float32).T.reshape(-1)
  q_cm = jnp.pad(q_coords.astype(jnp.float32),
                 ((0, _MP - M_QUERIES), (0, 0))).T.reshape(-1)
  idxp = idx.reshape(-1)
  flat = _sc_gather(pts_cm, q_cm, idxp)
  sT = flat.reshape(3 * K_SUPPORT, _MP)

  cy_a = 0.5 / (cy_params * cy_params)
  disk_a = 0.5 / (disk_params[:, 0] * disk_params[:, 0])
  disk_b = 0.5 / (disk_params[:, 1] * disk_params[:, 1])
  params = jnp.concatenate(
      [cy_a, disk_a, disk_b, cone_params[:, 0], cone_params[:, 1]]
  ).astype(jnp.float32)

  w = jnp.repeat(lambdas.astype(jnp.float32).T, K_SUPPORT, axis=1)
  return _tc_compute(sT, w, params)


# speedup vs baseline: 8.2589x; 1.0564x over previous
import functools

import jax
import jax.numpy as jnp
from jax import lax
from jax.experimental import pallas as pl
from jax.experimental.pallas import tpu as pltpu
from jax.experimental.pallas import tpu_sc as plsc

N_POINTS = 100000
M_QUERIES = 50000
K_SUPPORT = 16
G_CY = 8
G_DISK = 8
G_CONE = 8
NUM_GIBS = G_CY + G_DISK + G_CONE
NUM_OBS = 16
KERNEL_REACH = 0.3
REACH2 = KERNEL_REACH * KERNEL_REACH

_SC_CORES = 2
_SC_SUBCORES = 16
_NW = _SC_CORES * _SC_SUBCORES

_BM = 1024
_MP = 50176
_BP = _MP * K_SUPPORT
_B_PER_W = _BP // _NW
_Q_PER_W = _MP // _NW
_CHQ = 392
_CHUNK = _CHQ * K_SUPPORT
_N_CHUNKS = _Q_PER_W // _CHQ
_B_REAL = M_QUERIES * K_SUPPORT
_TAIL = _B_REAL - (_NW - 1) * _B_PER_W - (_N_CHUNKS - 1) * _CHUNK


def _sc_gather_body(pts_cm, idx_hbm, out_hbm,
                    tab_v, idx0, idx1, kt0, kt1, sem_i, sem_o):
  wid = lax.axis_index("s") * _SC_CORES + lax.axis_index("c")
  qbase = wid * _Q_PER_W
  ebase = wid * _B_PER_W
  iota = lax.iota(jnp.int32, 16)
  idx_bufs = (idx0, idx1)
  kt_bufs = (kt0, kt1)

  def fire_idx(c):
    buf = idx_bufs[c % 2]
    if c == _N_CHUNKS - 1:
      @pl.when(wid == _NW - 1)
      def _():
        pltpu.async_copy(idx_hbm.at[pl.ds(ebase + c * _CHUNK, _TAIL)],
                         buf.at[pl.ds(0, _TAIL)], sem_i)

      @pl.when(wid < _NW - 1)
      def _():
        pltpu.async_copy(idx_hbm.at[pl.ds(ebase + c * _CHUNK, _CHUNK)],
                         buf, sem_i)
    else:
      pltpu.async_copy(idx_hbm.at[pl.ds(ebase + c * _CHUNK, _CHUNK)],
                       buf, sem_i)

  def wait_idx(c):
    buf = idx_bufs[c % 2]
    if c == _N_CHUNKS - 1:
      @pl.when(wid == _NW - 1)
      def _():
        pltpu.make_async_copy(idx_hbm.at[pl.ds(ebase + c * _CHUNK, _TAIL)],
                              buf.at[pl.ds(0, _TAIL)], sem_i).wait()

      @pl.when(wid < _NW - 1)
      def _():
        pltpu.make_async_copy(idx_hbm.at[pl.ds(ebase + c * _CHUNK, _CHUNK)],
                              buf, sem_i).wait()
    else:
      pltpu.make_async_copy(idx_hbm.at[pl.ds(ebase + c * _CHUNK, _CHUNK)],
                            buf, sem_i).wait()

  for ci in range(3):
    pltpu.sync_copy(pts_cm.at[pl.ds(ci * N_POINTS, N_POINTS)], tab_v)
    fire_idx(0)
    out_cps = {}
    for c in range(_N_CHUNKS):
      wait_idx(c)
      if c + 1 < _N_CHUNKS:
        fire_idx(c + 1)
      if c - 2 in out_cps:
        for cp in out_cps.pop(c - 2):
          cp.wait()
      ktb = kt_bufs[c % 2]
      idxb = idx_bufs[c % 2]

      def gather_q(qidx, idxb, ktb):
        iv = idxb[pl.ds(qidx * 16, 16)]
        s = plsc.load_gather(tab_v, [iv])
        plsc.store_scatter(ktb, [iota, jnp.full((16,), qidx, jnp.int32)], s)

      def body(jq, _, idxb=idxb, ktb=ktb):
        for jj in range(16):
          gather_q(jq * 16 + jj, idxb, ktb)
        return ()

      lax.fori_loop(0, _CHQ // 16, body, (), unroll=2)
      for jj in range(_CHQ - (_CHQ // 16) * 16):
        gather_q((_CHQ // 16) * 16 + jj, idxb, ktb)
      cps = []
      for k in range(K_SUPPORT):
        off = (ci * K_SUPPORT + k) * _MP + qbase + c * _CHQ
        cps.append(pltpu.async_copy(ktb.at[k], out_hbm.at[pl.ds(off, _CHQ)],
                                    sem_o))
      out_cps[c] = cps
    for cl in out_cps.values():
      for cp in cl:
        cp.wait()


def _sc_gather(pts_cm, idx_flat):
  mesh = plsc.VectorSubcoreMesh(core_axis_name="c", subcore_axis_name="s")
  return pl.kernel(
      _sc_gather_body,
      out_type=jax.ShapeDtypeStruct((3 * K_SUPPORT * _MP,), jnp.float32),
      mesh=mesh,
      scratch_types=[
          pltpu.VMEM((N_POINTS,), jnp.float32),
          pltpu.VMEM((_CHUNK,), jnp.int32),
          pltpu.VMEM((_CHUNK,), jnp.int32),
          pltpu.VMEM((K_SUPPORT, _CHQ), jnp.float32),
          pltpu.VMEM((K_SUPPORT, _CHQ), jnp.float32),
          pltpu.SemaphoreType.DMA,
          pltpu.SemaphoreType.DMA,
      ],
      compiler_params=pltpu.CompilerParams(
          use_tc_tiling_on_sc=False, needs_layout_passes=False),
  )(pts_cm, idx_flat)


def _tc_body(s_ref, q_ref, w_ref, par_ref, out_ref, v_ref):
  K = K_SUPPORT
  rx = s_ref[0:K, :] - q_ref[0:1, :]
  ry = s_ref[K:2 * K, :] - q_ref[1:2, :]
  rz = s_ref[2 * K:3 * K, :] - q_ref[2:3, :]
  x2y2 = rx * rx + ry * ry
  z2 = rz * rz
  mask = x2y2 + z2 <= REACH2
  zero = jnp.zeros_like(x2y2)
  nx = x2y2 * (-0.5)

  for g in range(G_CY):
    a = par_ref[g]
    v_ref[pl.ds(g * K, K), :] = jnp.where(
        mask, jnp.exp(x2y2 * (-a)), zero)
  for g in range(G_DISK):
    a = par_ref[G_CY + g]
    b = par_ref[G_CY + G_DISK + g]
    v_ref[pl.ds((G_CY + g) * K, K), :] = jnp.where(
        mask, jnp.exp(-(x2y2 * a + z2 * b)), zero)
  for g in range(G_CONE):
    cr = par_ref[G_CY + 2 * G_DISK + g]
    ci = par_ref[G_CY + 2 * G_DISK + G_CONE + g]
    r = jnp.maximum(cr * (1.0 + ci * rz), 1e-3)
    inv = pl.reciprocal(r, approx=True)
    v_ref[pl.ds((G_CY + G_DISK + g) * K, K), :] = jnp.where(
        mask, jnp.exp(nx * (inv * inv)), zero)

  out_ref[...] = lax.dot_general(
      v_ref[...], w_ref[...],
      dimension_numbers=(((0,), (1,)), ((), ())),
      preferred_element_type=jnp.float32)


def _tc_compute(sT, q8, w, params):
  return pl.pallas_call(
      _tc_body,
      grid=(_MP // _BM,),
      in_specs=[
          pl.BlockSpec((3 * K_SUPPORT, _BM), lambda i: (0, i)),
          pl.BlockSpec((8, _BM), lambda i: (0, i)),
          pl.BlockSpec((NUM_OBS, NUM_GIBS * K_SUPPORT), lambda i: (0, 0)),
          pl.BlockSpec(memory_space=pltpu.SMEM),
      ],
      out_specs=pl.BlockSpec((_BM, NUM_OBS), lambda i: (i, 0)),
      out_shape=jax.ShapeDtypeStruct((M_QUERIES, NUM_OBS), jnp.float32),
      scratch_shapes=[pltpu.VMEM((NUM_GIBS * K_SUPPORT, _BM), jnp.float32)],
  )(sT, q8, w, params)


def kernel(points, q_coords, support_idxs, cy_params, disk_params,
           cone_params, lambdas):
  idx = support_idxs.astype(jnp.int32)
  pts_cm = points.astype(jnp.float32).T.reshape(-1)
  q8 = jnp.pad(q_coords.astype(jnp.float32).T,
               ((0, 5), (0, _MP - M_QUERIES)))
  idxp = idx.reshape(-1)
  flat = _sc_gather(pts_cm, idxp)
  sT = flat.reshape(3 * K_SUPPORT, _MP)

  cy_a = 0.5 / (cy_params * cy_params)
  disk_a = 0.5 / (disk_params[:, 0] * disk_params[:, 0])
  disk_b = 0.5 / (disk_params[:, 1] * disk_params[:, 1])
  params = jnp.concatenate(
      [cy_a, disk_a, disk_b, cone_params[:, 0], cone_params[:, 1]]
  ).astype(jnp.float32)

  w = jnp.repeat(lambdas.astype(jnp.float32).T, K_SUPPORT, axis=1)
  return _tc_compute(sT, q8, w, params)


# speedup vs baseline: 8.6990x; 1.0533x over previous
import functools

import jax
import jax.numpy as jnp
from jax import lax
from jax.experimental import pallas as pl
from jax.experimental.pallas import tpu as pltpu
from jax.experimental.pallas import tpu_sc as plsc

N_POINTS = 100000
M_QUERIES = 50000
K_SUPPORT = 16
G_CY = 8
G_DISK = 8
G_CONE = 8
NUM_GIBS = G_CY + G_DISK + G_CONE
NUM_OBS = 16
KERNEL_REACH = 0.3
REACH2 = KERNEL_REACH * KERNEL_REACH

_SC_CORES = 2
_SC_SUBCORES = 16
_NW = _SC_CORES * _SC_SUBCORES

_BM = 1792
_MP = 50176
_BP = _MP * K_SUPPORT
_B_PER_W = _BP // _NW
_Q_PER_W = _MP // _NW
_CHQ = 392
_CHUNK = _CHQ * K_SUPPORT
_N_CHUNKS = _Q_PER_W // _CHQ
_B_REAL = M_QUERIES * K_SUPPORT
_TAIL = _B_REAL - (_NW - 1) * _B_PER_W - (_N_CHUNKS - 1) * _CHUNK


def _sc_gather_body(pts_cm, idx_hbm, out_hbm,
                    tab_v, idx0, idx1, kt0, kt1, sem_i, sem_o):
  wid = lax.axis_index("s") * _SC_CORES + lax.axis_index("c")
  qbase = wid * _Q_PER_W
  ebase = wid * _B_PER_W
  iota = lax.iota(jnp.int32, 16)
  idx_bufs = (idx0, idx1)
  kt_bufs = (kt0, kt1)

  def fire_idx(c):
    buf = idx_bufs[c % 2]
    if c == _N_CHUNKS - 1:
      @pl.when(wid == _NW - 1)
      def _():
        pltpu.async_copy(idx_hbm.at[pl.ds(ebase + c * _CHUNK, _TAIL)],
                         buf.at[pl.ds(0, _TAIL)], sem_i)

      @pl.when(wid < _NW - 1)
      def _():
        pltpu.async_copy(idx_hbm.at[pl.ds(ebase + c * _CHUNK, _CHUNK)],
                         buf, sem_i)
    else:
      pltpu.async_copy(idx_hbm.at[pl.ds(ebase + c * _CHUNK, _CHUNK)],
                       buf, sem_i)

  def wait_idx(c):
    buf = idx_bufs[c % 2]
    if c == _N_CHUNKS - 1:
      @pl.when(wid == _NW - 1)
      def _():
        pltpu.make_async_copy(idx_hbm.at[pl.ds(ebase + c * _CHUNK, _TAIL)],
                              buf.at[pl.ds(0, _TAIL)], sem_i).wait()

      @pl.when(wid < _NW - 1)
      def _():
        pltpu.make_async_copy(idx_hbm.at[pl.ds(ebase + c * _CHUNK, _CHUNK)],
                              buf, sem_i).wait()
    else:
      pltpu.make_async_copy(idx_hbm.at[pl.ds(ebase + c * _CHUNK, _CHUNK)],
                            buf, sem_i).wait()

  for ci in range(3):
    pltpu.sync_copy(pts_cm.at[pl.ds(ci * N_POINTS, N_POINTS)], tab_v)
    fire_idx(0)
    out_cps = {}
    for c in range(_N_CHUNKS):
      wait_idx(c)
      if c + 1 < _N_CHUNKS:
        fire_idx(c + 1)
      if c - 2 in out_cps:
        for cp in out_cps.pop(c - 2):
          cp.wait()
      ktb = kt_bufs[c % 2]
      idxb = idx_bufs[c % 2]

      def gather_q(qidx, idxb, ktb):
        iv = idxb[pl.ds(qidx * 16, 16)]
        s = plsc.load_gather(tab_v, [iv])
        plsc.store_scatter(ktb, [iota, jnp.full((16,), qidx, jnp.int32)], s)

      def body(jq, _, idxb=idxb, ktb=ktb):
        for jj in range(16):
          gather_q(jq * 16 + jj, idxb, ktb)
        return ()

      lax.fori_loop(0, _CHQ // 16, body, (), unroll=2)
      for jj in range(_CHQ - (_CHQ // 16) * 16):
        gather_q((_CHQ // 16) * 16 + jj, idxb, ktb)
      cps = []
      for k in range(K_SUPPORT):
        off = (ci * K_SUPPORT + k) * _MP + qbase + c * _CHQ
        cps.append(pltpu.async_copy(ktb.at[k], out_hbm.at[pl.ds(off, _CHQ)],
                                    sem_o))
      out_cps[c] = cps
    for cl in out_cps.values():
      for cp in cl:
        cp.wait()


def _sc_gather(pts_cm, idx_flat):
  mesh = plsc.VectorSubcoreMesh(core_axis_name="c", subcore_axis_name="s")
  return pl.kernel(
      _sc_gather_body,
      out_type=jax.ShapeDtypeStruct((3 * K_SUPPORT * _MP,), jnp.float32),
      mesh=mesh,
      scratch_types=[
          pltpu.VMEM((N_POINTS,), jnp.float32),
          pltpu.VMEM((_CHUNK,), jnp.int32),
          pltpu.VMEM((_CHUNK,), jnp.int32),
          pltpu.VMEM((K_SUPPORT, _CHQ), jnp.float32),
          pltpu.VMEM((K_SUPPORT, _CHQ), jnp.float32),
          pltpu.SemaphoreType.DMA,
          pltpu.SemaphoreType.DMA,
      ],
      compiler_params=pltpu.CompilerParams(
          use_tc_tiling_on_sc=False, needs_layout_passes=False),
  )(pts_cm, idx_flat)


def _tc_body(s_ref, q_ref, w_ref, par_ref, out_ref, v_ref):
  K = K_SUPPORT
  rx = s_ref[0:K, :] - q_ref[0:1, :]
  ry = s_ref[K:2 * K, :] - q_ref[1:2, :]
  rz = s_ref[2 * K:3 * K, :] - q_ref[2:3, :]
  x2y2 = rx * rx + ry * ry
  z2 = rz * rz
  mask = x2y2 + z2 <= REACH2
  zero = jnp.zeros_like(x2y2)
  nx = x2y2 * (-0.5)

  for g in range(G_CY):
    a = par_ref[g]
    v_ref[pl.ds(g * K, K), :] = jnp.where(
        mask, jnp.exp(x2y2 * (-a)), zero)
  for g in range(G_DISK):
    a = par_ref[G_CY + g]
    b = par_ref[G_CY + G_DISK + g]
    v_ref[pl.ds((G_CY + g) * K, K), :] = jnp.where(
        mask, jnp.exp(-(x2y2 * a + z2 * b)), zero)
  for g in range(G_CONE):
    cr = par_ref[G_CY + 2 * G_DISK + g]
    ci = par_ref[G_CY + 2 * G_DISK + G_CONE + g]
    r = jnp.maximum(cr * (1.0 + ci * rz), 1e-3)
    inv = pl.reciprocal(r, approx=True)
    v_ref[pl.ds((G_CY + G_DISK + g) * K, K), :] = jnp.where(
        mask, jnp.exp(nx * (inv * inv)), zero)

  out_ref[...] = lax.dot_general(
      v_ref[...], w_ref[...],
      dimension_numbers=(((0,), (1,)), ((), ())),
      preferred_element_type=jnp.float32)


def _tc_compute(sT, q8, w, params):
  return pl.pallas_call(
      _tc_body,
      grid=(_MP // _BM,),
      in_specs=[
          pl.BlockSpec((3 * K_SUPPORT, _BM), lambda i: (0, i)),
          pl.BlockSpec((8, _BM), lambda i: (0, i)),
          pl.BlockSpec((NUM_OBS, NUM_GIBS * K_SUPPORT), lambda i: (0, 0)),
          pl.BlockSpec(memory_space=pltpu.SMEM),
      ],
      out_specs=pl.BlockSpec((_BM, NUM_OBS), lambda i: (i, 0)),
      out_shape=jax.ShapeDtypeStruct((M_QUERIES, NUM_OBS), jnp.float32),
      scratch_shapes=[pltpu.VMEM((NUM_GIBS * K_SUPPORT, _BM), jnp.float32)],
  )(sT, q8, w, params)


def kernel(points, q_coords, support_idxs, cy_params, disk_params,
           cone_params, lambdas):
  idx = support_idxs.astype(jnp.int32)
  pts_cm = points.astype(jnp.float32).T.reshape(-1)
  q8 = jnp.pad(q_coords.astype(jnp.float32).T,
               ((0, 5), (0, _MP - M_QUERIES)))
  idxp = idx.reshape(-1)
  flat = _sc_gather(pts_cm, idxp)
  sT = flat.reshape(3 * K_SUPPORT, _MP)

  cy_a = 0.5 / (cy_params * cy_params)
  disk_a = 0.5 / (disk_params[:, 0] * disk_params[:, 0])
  disk_b = 0.5 / (disk_params[:, 1] * disk_params[:, 1])
  params = jnp.concatenate(
      [cy_a, disk_a, disk_b, cone_params[:, 0], cone_params[:, 1]]
  ).astype(jnp.float32)

  w = jnp.repeat(lambdas.astype(jnp.float32).T, K_SUPPORT, axis=1)
  return _tc_compute(sT, q8, w, params)


# speedup vs baseline: 8.9501x; 1.0289x over previous
import functools

import jax
import jax.numpy as jnp
from jax import lax
from jax.experimental import pallas as pl
from jax.experimental.pallas import tpu as pltpu
from jax.experimental.pallas import tpu_sc as plsc

N_POINTS = 100000
M_QUERIES = 50000
K_SUPPORT = 16
G_CY = 8
G_DISK = 8
G_CONE = 8
NUM_GIBS = G_CY + G_DISK + G_CONE
NUM_OBS = 16
KERNEL_REACH = 0.3
REACH2 = KERNEL_REACH * KERNEL_REACH

_SC_CORES = 2
_SC_SUBCORES = 16
_NW = _SC_CORES * _SC_SUBCORES

_BM = 3584
_MP = 50176
_BP = _MP * K_SUPPORT
_B_PER_W = _BP // _NW
_Q_PER_W = _MP // _NW
_CHQ = 392
_CHUNK = _CHQ * K_SUPPORT
_N_CHUNKS = _Q_PER_W // _CHQ
_B_REAL = M_QUERIES * K_SUPPORT
_TAIL = _B_REAL - (_NW - 1) * _B_PER_W - (_N_CHUNKS - 1) * _CHUNK


def _sc_gather_body(pts_cm, idx_hbm, out_hbm,
                    tab_v, idx0, idx1, kt0, kt1, sem_i, sem_o):
  wid = lax.axis_index("s") * _SC_CORES + lax.axis_index("c")
  qbase = wid * _Q_PER_W
  ebase = wid * _B_PER_W
  iota = lax.iota(jnp.int32, 16)
  idx_bufs = (idx0, idx1)
  kt_bufs = (kt0, kt1)

  def fire_idx(c):
    buf = idx_bufs[c % 2]
    if c == _N_CHUNKS - 1:
      @pl.when(wid == _NW - 1)
      def _():
        pltpu.async_copy(idx_hbm.at[pl.ds(ebase + c * _CHUNK, _TAIL)],
                         buf.at[pl.ds(0, _TAIL)], sem_i)

      @pl.when(wid < _NW - 1)
      def _():
        pltpu.async_copy(idx_hbm.at[pl.ds(ebase + c * _CHUNK, _CHUNK)],
                         buf, sem_i)
    else:
      pltpu.async_copy(idx_hbm.at[pl.ds(ebase + c * _CHUNK, _CHUNK)],
                       buf, sem_i)

  def wait_idx(c):
    buf = idx_bufs[c % 2]
    if c == _N_CHUNKS - 1:
      @pl.when(wid == _NW - 1)
      def _():
        pltpu.make_async_copy(idx_hbm.at[pl.ds(ebase + c * _CHUNK, _TAIL)],
                              buf.at[pl.ds(0, _TAIL)], sem_i).wait()

      @pl.when(wid < _NW - 1)
      def _():
        pltpu.make_async_copy(idx_hbm.at[pl.ds(ebase + c * _CHUNK, _CHUNK)],
                              buf, sem_i).wait()
    else:
      pltpu.make_async_copy(idx_hbm.at[pl.ds(ebase + c * _CHUNK, _CHUNK)],
                            buf, sem_i).wait()

  for ci in range(3):
    pltpu.sync_copy(pts_cm.at[pl.ds(ci * N_POINTS, N_POINTS)], tab_v)
    fire_idx(0)
    out_cps = {}
    for c in range(_N_CHUNKS):
      wait_idx(c)
      if c + 1 < _N_CHUNKS:
        fire_idx(c + 1)
      if c - 2 in out_cps:
        for cp in out_cps.pop(c - 2):
          cp.wait()
      ktb = kt_bufs[c % 2]
      idxb = idx_bufs[c % 2]

      def gather_q(qidx, idxb, ktb):
        iv = idxb[pl.ds(qidx * 16, 16)]
        s = plsc.load_gather(tab_v, [iv])
        plsc.store_scatter(ktb, [iota, jnp.full((16,), qidx, jnp.int32)], s)

      def body(jq, _, idxb=idxb, ktb=ktb):
        for jj in range(16):
          gather_q(jq * 16 + jj, idxb, ktb)
        return ()

      lax.fori_loop(0, _CHQ // 16, body, (), unroll=2)
      for jj in range(_CHQ - (_CHQ // 16) * 16):
        gather_q((_CHQ // 16) * 16 + jj, idxb, ktb)
      cps = []
      for k in range(K_SUPPORT):
        off = (ci * K_SUPPORT + k) * _MP + qbase + c * _CHQ
        cps.append(pltpu.async_copy(ktb.at[k], out_hbm.at[pl.ds(off, _CHQ)],
                                    sem_o))
      out_cps[c] = cps
    for cl in out_cps.values():
      for cp in cl:
        cp.wait()


def _sc_gather(pts_cm, idx_flat):
  mesh = plsc.VectorSubcoreMesh(core_axis_name="c", subcore_axis_name="s")
  return pl.kernel(
      _sc_gather_body,
      out_type=jax.ShapeDtypeStruct((3 * K_SUPPORT * _MP,), jnp.float32),
      mesh=mesh,
      scratch_types=[
          pltpu.VMEM((N_POINTS,), jnp.float32),
          pltpu.VMEM((_CHUNK,), jnp.int32),
          pltpu.VMEM((_CHUNK,), jnp.int32),
          pltpu.VMEM((K_SUPPORT, _CHQ), jnp.float32),
          pltpu.VMEM((K_SUPPORT, _CHQ), jnp.float32),
          pltpu.SemaphoreType.DMA,
          pltpu.SemaphoreType.DMA,
      ],
      compiler_params=pltpu.CompilerParams(
          use_tc_tiling_on_sc=False, needs_layout_passes=False),
  )(pts_cm, idx_flat)


def _tc_body(s_ref, q_ref, w_ref, par_ref, out_ref, v_ref):
  K = K_SUPPORT
  rx = s_ref[0:K, :] - q_ref[0:1, :]
  ry = s_ref[K:2 * K, :] - q_ref[1:2, :]
  rz = s_ref[2 * K:3 * K, :] - q_ref[2:3, :]
  x2y2 = rx * rx + ry * ry
  z2 = rz * rz
  mask = x2y2 + z2 <= REACH2
  zero = jnp.zeros_like(x2y2)
  nx = x2y2 * (-0.5)

  for g in range(G_CY):
    a = par_ref[g]
    v_ref[pl.ds(g * K, K), :] = jnp.where(
        mask, jnp.exp(x2y2 * (-a)), zero)
  for g in range(G_DISK):
    a = par_ref[G_CY + g]
    b = par_ref[G_CY + G_DISK + g]
    v_ref[pl.ds((G_CY + g) * K, K), :] = jnp.where(
        mask, jnp.exp(-(x2y2 * a + z2 * b)), zero)
  for g in range(G_CONE):
    cr = par_ref[G_CY + 2 * G_DISK + g]
    ci = par_ref[G_CY + 2 * G_DISK + G_CONE + g]
    r = jnp.maximum(cr * (1.0 + ci * rz), 1e-3)
    inv = pl.reciprocal(r, approx=True)
    v_ref[pl.ds((G_CY + G_DISK + g) * K, K), :] = jnp.where(
        mask, jnp.exp(nx * (inv * inv)), zero)

  out_ref[...] = lax.dot_general(
      v_ref[...], w_ref[...],
      dimension_numbers=(((0,), (1,)), ((), ())),
      preferred_element_type=jnp.float32)


def _tc_compute(sT, q8, w, params):
  return pl.pallas_call(
      _tc_body,
      grid=(_MP // _BM,),
      in_specs=[
          pl.BlockSpec((3 * K_SUPPORT, _BM), lambda i: (0, i)),
          pl.BlockSpec((8, _BM), lambda i: (0, i)),
          pl.BlockSpec((NUM_OBS, NUM_GIBS * K_SUPPORT), lambda i: (0, 0)),
          pl.BlockSpec(memory_space=pltpu.SMEM),
      ],
      out_specs=pl.BlockSpec((_BM, NUM_OBS), lambda i: (i, 0)),
      out_shape=jax.ShapeDtypeStruct((M_QUERIES, NUM_OBS), jnp.float32),
      scratch_shapes=[pltpu.VMEM((NUM_GIBS * K_SUPPORT, _BM), jnp.float32)],
  )(sT, q8, w, params)


def kernel(points, q_coords, support_idxs, cy_params, disk_params,
           cone_params, lambdas):
  idx = support_idxs.astype(jnp.int32)
  pts_cm = points.astype(jnp.float32).T.reshape(-1)
  q8 = jnp.pad(q_coords.astype(jnp.float32).T,
               ((0, 5), (0, _MP - M_QUERIES)))
  idxp = idx.reshape(-1)
  flat = _sc_gather(pts_cm, idxp)
  sT = flat.reshape(3 * K_SUPPORT, _MP)

  cy_a = 0.5 / (cy_params * cy_params)
  disk_a = 0.5 / (disk_params[:, 0] * disk_params[:, 0])
  disk_b = 0.5 / (disk_params[:, 1] * disk_params[:, 1])
  params = jnp.concatenate(
      [cy_a, disk_a, disk_b, cone_params[:, 0], cone_params[:, 1]]
  ).astype(jnp.float32)

  w = jnp.repeat(lambdas.astype(jnp.float32).T, K_SUPPORT, axis=1)
  return _tc_compute(sT, q8, w, params)
